# Initial kernel scaffold; baseline (speedup 1.0000x reference)
import jax
import jax.numpy as jnp
from jax.experimental import pallas as pl


def kernel(x, edge_index, edge_weight, batch, W1, b1, W2, b2, W3, b3, L1w, L1b, L2w, L2b, L3w, L3b, L4w, L4b):
    raise NotImplementedError("write your pallas kernel here")


# speedup vs baseline: 17.2277x; 17.2277x over previous
import functools

import jax
import jax.numpy as jnp
from jax import lax
from jax.experimental import pallas as pl
from jax.experimental.pallas import tpu as pltpu
from jax.experimental.pallas import tpu_sc as plsc

N = 10000
E = 320000
F_IN = 128
H = 32
MH = 64
C_OUT = 10
G = 16

NC = 2
NS = 16
NW = NC * NS
CHUNK = 128
CPW = 80
NCHUNK = NW * CPW
E_PAD = NCHUNK * CHUNK
N_PAD = 10240
RPT = N_PAD // NS

_mesh = plsc.VectorSubcoreMesh(core_axis_name="c", subcore_axis_name="s",
                               num_cores=NC, num_subcores=NS)
_sc_params = pltpu.CompilerParams(use_tc_tiling_on_sc=False)


def _deg_body(colm, wm, out_hbm, deg_sh, col_v, w_v, zbuf):
    c = lax.axis_index("c")
    s = lax.axis_index("s")
    wid = s * NC + c
    lo = pl.multiple_of(wid * CPW, 8)
    pltpu.sync_copy(colm.at[pl.ds(lo, CPW)], col_v)
    pltpu.sync_copy(wm.at[pl.ds(lo, CPW)], w_v)

    zeros16 = jnp.zeros((16,), jnp.float32)

    def zbody(i, carry):
        zbuf[pl.ds(i * 16, 16)] = zeros16
        return carry

    lax.fori_loop(0, RPT // 16, zbody, 0)
    pltpu.sync_copy(zbuf, deg_sh.at[pl.ds(s * RPT, RPT)])
    plsc.subcore_barrier()

    def chunk_body(j, carry):
        pltpu.sync_copy(w_v.at[j], deg_sh.at[col_v.at[j]], add=True)
        return carry

    lax.fori_loop(0, CPW, chunk_body, 0)
    plsc.subcore_barrier()
    pltpu.sync_copy(deg_sh.at[pl.ds(s * RPT, RPT)],
                    out_hbm.at[c, pl.ds(s * RPT, RPT)])


_deg_call = functools.partial(
    pl.kernel,
    _deg_body,
    out_type=jax.ShapeDtypeStruct((NC, N_PAD), jnp.float32),
    mesh=_mesh,
    compiler_params=_sc_params,
    scratch_types=[
        pltpu.VMEM_SHARED((N_PAD,), jnp.float32),
        pltpu.VMEM((CPW, CHUNK), jnp.int32),
        pltpu.VMEM((CPW, CHUNK), jnp.float32),
        pltpu.VMEM((RPT,), jnp.float32),
    ],
)()


def _layer_body(y_hbm, rowm, colm, wm, out_hbm,
                acc_sh, row_v, col_v, w_v, buf, zbuf):
    c = lax.axis_index("c")
    s = lax.axis_index("s")
    wid = s * NC + c
    lo = pl.multiple_of(wid * CPW, 8)
    pltpu.sync_copy(rowm.at[pl.ds(lo, CPW)], row_v)
    pltpu.sync_copy(colm.at[pl.ds(lo, CPW)], col_v)
    pltpu.sync_copy(wm.at[pl.ds(lo, CPW)], w_v)

    zeros16 = jnp.zeros((16,), jnp.float32)

    def zbody(i, carry):
        zbuf[i, pl.ds(0, 16)] = zeros16
        zbuf[i, pl.ds(16, 16)] = zeros16
        return carry

    lax.fori_loop(0, RPT, zbody, 0)
    pltpu.sync_copy(zbuf, acc_sh.at[pl.ds(s * RPT, RPT)])
    plsc.subcore_barrier()

    def chunk_body(j, carry):
        pltpu.sync_copy(y_hbm.at[row_v.at[j]], buf)

        def scale(g, icarry):
            wv = w_v[j, pl.ds(g * 16, 16)]
            for l in range(16):
                e = g * 16 + l
                sw = wv[l]
                buf[e, pl.ds(0, 16)] = buf[e, pl.ds(0, 16)] * sw
                buf[e, pl.ds(16, 16)] = buf[e, pl.ds(16, 16)] * sw
            return icarry

        lax.fori_loop(0, CHUNK // 16, scale, 0)
        pltpu.sync_copy(buf, acc_sh.at[col_v.at[j]], add=True)
        return carry

    lax.fori_loop(0, CPW, chunk_body, 0)
    plsc.subcore_barrier()
    pltpu.sync_copy(acc_sh.at[pl.ds(s * RPT, RPT)],
                    out_hbm.at[c, pl.ds(s * RPT, RPT)])


_layer_call = functools.partial(
    pl.kernel,
    _layer_body,
    out_type=jax.ShapeDtypeStruct((NC, N_PAD, H), jnp.float32),
    mesh=_mesh,
    compiler_params=_sc_params,
    scratch_types=[
        pltpu.VMEM_SHARED((N_PAD, H), jnp.float32),
        pltpu.VMEM((CPW, CHUNK), jnp.int32),
        pltpu.VMEM((CPW, CHUNK), jnp.int32),
        pltpu.VMEM((CPW, CHUNK), jnp.float32),
        pltpu.VMEM((CHUNK, H), jnp.float32),
        pltpu.VMEM((RPT, H), jnp.float32),
    ],
)()


def _tc_first(x_ref, w1_ref, degp_ref, y_ref):
    deg = jnp.sum(degp_ref[:, :N], axis=0) + 1.0
    dinv = lax.rsqrt(deg)
    xw = jnp.dot(x_ref[...], w1_ref[...], preferred_element_type=jnp.float32)
    y_ref[...] = xw * dinv[:, None]


def _tc_mid(a_ref, y_ref, degp_ref, b_ref, w_ref, out_ref):
    deg = jnp.sum(degp_ref[:, :N], axis=0) + 1.0
    dinv = lax.rsqrt(deg)
    asum = a_ref[0, :N] + a_ref[1, :N] + y_ref[...]
    h = jnp.maximum(asum * dinv[:, None] + b_ref[...], 0.0)
    out_ref[...] = jnp.dot(
        h, w_ref[...], preferred_element_type=jnp.float32) * dinv[:, None]


def _tc_final(a_ref, y_ref, degp_ref, b3_ref, batch_ref,
              l1w, l1b, l2w, l2b, l3w, l3b, l4w, l4b, out_ref):
    deg = jnp.sum(degp_ref[:, :N], axis=0) + 1.0
    dinv = lax.rsqrt(deg)
    h = (a_ref[0, :N] + a_ref[1, :N] + y_ref[...]) * dinv[:, None] + b3_ref[...]
    onehot = (batch_ref[...] == lax.broadcasted_iota(jnp.int32, (1, G), 1))
    onehot = onehot.astype(jnp.float32)
    sums = lax.dot_general(onehot, h, (((0,), (0,)), ((), ())),
                           preferred_element_type=jnp.float32)
    counts = jnp.sum(onehot, axis=0)
    pooled = sums / jnp.maximum(counts, 1.0)[:, None]
    o = jnp.maximum(jnp.dot(pooled, l1w[...],
                            preferred_element_type=jnp.float32) + l1b[...], 0.0)
    o = jnp.maximum(jnp.dot(o, l2w[...],
                            preferred_element_type=jnp.float32) + l2b[...], 0.0)
    o = jnp.maximum(jnp.dot(o, l3w[...],
                            preferred_element_type=jnp.float32) + l3b[...], 0.0)
    out_ref[...] = jnp.dot(o, l4w[...],
                           preferred_element_type=jnp.float32) + l4b[...]


_first_call = pl.pallas_call(
    _tc_first, out_shape=jax.ShapeDtypeStruct((N, H), jnp.float32))
_mid_call = pl.pallas_call(
    _tc_mid, out_shape=jax.ShapeDtypeStruct((N, H), jnp.float32))
_final_call = pl.pallas_call(
    _tc_final, out_shape=jax.ShapeDtypeStruct((G, C_OUT), jnp.float32))


def kernel(x, edge_index, edge_weight, batch,
           W1, b1, W2, b2, W3, b3,
           L1w, L1b, L2w, L2b, L3w, L3b, L4w, L4b):
    pad_i = jnp.zeros((E_PAD - E,), jnp.int32)
    pad_f = jnp.zeros((E_PAD - E,), jnp.float32)
    rowm = jnp.concatenate([edge_index[0], pad_i]).reshape(NCHUNK, CHUNK)
    colm = jnp.concatenate([edge_index[1], pad_i]).reshape(NCHUNK, CHUNK)
    wm = jnp.concatenate([edge_weight, pad_f]).reshape(NCHUNK, CHUNK)

    degp = _deg_call(colm, wm)
    y1 = _first_call(x, W1, degp)
    a1 = _layer_call(y1, rowm, colm, wm)
    y2 = _mid_call(a1, y1, degp, b1, W2)
    a2 = _layer_call(y2, rowm, colm, wm)
    y3 = _mid_call(a2, y2, degp, b2, W3)
    a3 = _layer_call(y3, rowm, colm, wm)
    return _final_call(a3, y3, degp, b3, batch.reshape(N, 1),
                       L1w, L1b, L2w, L2b, L3w, L3b, L4w, L4b)
